# Initial kernel scaffold; baseline (speedup 1.0000x reference)
import jax
import jax.numpy as jnp
from jax.experimental import pallas as pl


def kernel(x, W, b):
    raise NotImplementedError("write your pallas kernel here")


# speedup vs baseline: 3.1724x; 3.1724x over previous
import math

import jax
import jax.numpy as jnp
from jax.experimental import pallas as pl
from jax.experimental.pallas import tpu as pltpu

_TEMP_INV = 1.0 / math.e
_N_EXPERTS = 8
_N_ACTIVE = 5


def _gate_body(x_ref, w_ref, b_ref, o_ref):
    x = x_ref[...]
    w = w_ref[...]
    s = jax.lax.dot_general(
        x, w, (((1,), (1,)), ((), ())),
        preferred_element_type=jnp.float32)
    s = (s + b_ref[...]) * _TEMP_INV
    m = jnp.max(s, axis=1, keepdims=True)
    e = jnp.exp(s - m)
    z = jnp.sum(e, axis=1, keepdims=True)
    p = e / z

    lane = jax.lax.broadcasted_iota(jnp.int32, p.shape, 1)
    cols = []
    for i in range(_N_EXPERTS):
        pi = p[:, i:i + 1]
        gt = (p > pi).astype(jnp.int32)
        tie = ((p == pi) & (lane < i)).astype(jnp.int32)
        rank = jnp.sum(gt + tie, axis=1, keepdims=True)
        cols.append((rank < _N_ACTIVE).astype(jnp.float32))
    keep = jnp.concatenate(cols, axis=1)

    kept = p * keep
    denom = jnp.sum(kept, axis=1, keepdims=True) + 1e-8
    o_ref[...] = kept / denom


def kernel(x, W, b):
    n, d = x.shape
    rows = 1024
    grid = (n // rows,)
    b2 = b.reshape(1, _N_EXPERTS)
    return pl.pallas_call(
        _gate_body,
        grid=grid,
        in_specs=[
            pl.BlockSpec((rows, d), lambda i: (i, 0)),
            pl.BlockSpec((_N_EXPERTS, d), lambda i: (0, 0)),
            pl.BlockSpec((1, _N_EXPERTS), lambda i: (0, 0)),
        ],
        out_specs=pl.BlockSpec((rows, _N_EXPERTS), lambda i: (i, 0)),
        out_shape=jax.ShapeDtypeStruct((n, _N_EXPERTS), jnp.float32),
    )(x, W, b2)


# speedup vs baseline: 18.3233x; 5.7758x over previous
import math

import jax
import jax.numpy as jnp
from jax.experimental import pallas as pl

_TEMP_INV = 1.0 / math.e
_N_EXPERTS = 8
_N_ACTIVE = 5


def _gate_body(x_ref, w_ref, b_ref, o_ref):
    x = x_ref[...]
    w = w_ref[...]
    s = jax.lax.dot_general(
        w, x, (((1,), (1,)), ((), ())),
        preferred_element_type=jnp.float32)
    s = (s + b_ref[...]) * _TEMP_INV
    m = jnp.max(s, axis=0, keepdims=True)
    e = jnp.exp(s - m)
    z = jnp.sum(e, axis=0, keepdims=True)
    p = e / z

    rows = []
    for i in range(_N_EXPERTS):
        pi = p[i:i + 1, :]
        gt = (p > pi).astype(jnp.float32)
        tie = (p[:i] == pi).astype(jnp.float32) if i else None
        rank = jnp.sum(gt, axis=0, keepdims=True)
        if tie is not None:
            rank = rank + jnp.sum(tie, axis=0, keepdims=True)
        rows.append((rank < _N_ACTIVE).astype(jnp.float32))
    keep = jnp.concatenate(rows, axis=0)

    kept = p * keep
    denom = jnp.sum(kept, axis=0, keepdims=True) + 1e-8
    o_ref[...] = kept / denom


def kernel(x, W, b):
    n, d = x.shape
    rows = 2048
    grid = (n // rows,)
    b2 = b.reshape(_N_EXPERTS, 1)
    out_t = pl.pallas_call(
        _gate_body,
        grid=grid,
        in_specs=[
            pl.BlockSpec((rows, d), lambda i: (i, 0)),
            pl.BlockSpec((_N_EXPERTS, d), lambda i: (0, 0)),
            pl.BlockSpec((_N_EXPERTS, 1), lambda i: (0, 0)),
        ],
        out_specs=pl.BlockSpec((_N_EXPERTS, rows), lambda i: (0, i)),
        out_shape=jax.ShapeDtypeStruct((_N_EXPERTS, n), jnp.float32),
    )(x, W, b2)
    return out_t.T


# speedup vs baseline: 19.1825x; 1.0469x over previous
import math

import jax
import jax.numpy as jnp
from jax.experimental import pallas as pl

_TEMP_INV = 1.0 / math.e
_N_EXPERTS = 8
_N_ACTIVE = 5


def _gate_body(x_ref, w_ref, b_ref, o_ref):
    x = x_ref[...]
    w = w_ref[...]
    s = jax.lax.dot_general(
        w, x, (((1,), (1,)), ((), ())),
        preferred_element_type=jnp.float32)
    s = (s + b_ref[...]) * _TEMP_INV
    m = jnp.max(s, axis=0, keepdims=True)
    e = jnp.exp(s - m)
    z = jnp.sum(e, axis=0, keepdims=True)
    p = e / z

    rows = []
    for i in range(_N_EXPERTS):
        pi = p[i:i + 1, :]
        gt = (p > pi).astype(jnp.float32)
        tie = (p[:i] == pi).astype(jnp.float32) if i else None
        rank = jnp.sum(gt, axis=0, keepdims=True)
        if tie is not None:
            rank = rank + jnp.sum(tie, axis=0, keepdims=True)
        rows.append((rank < _N_ACTIVE).astype(jnp.float32))
    keep = jnp.concatenate(rows, axis=0)

    kept = p * keep
    denom = jnp.sum(kept, axis=0, keepdims=True) + 1e-8
    o_ref[...] = kept / denom


def kernel(x, W, b):
    n, d = x.shape
    rows = 4096
    grid = (n // rows,)
    b2 = b.reshape(_N_EXPERTS, 1)
    out_t = pl.pallas_call(
        _gate_body,
        grid=grid,
        in_specs=[
            pl.BlockSpec((rows, d), lambda i: (i, 0)),
            pl.BlockSpec((_N_EXPERTS, d), lambda i: (0, 0)),
            pl.BlockSpec((_N_EXPERTS, 1), lambda i: (0, 0)),
        ],
        out_specs=pl.BlockSpec((_N_EXPERTS, rows), lambda i: (0, i)),
        out_shape=jax.ShapeDtypeStruct((_N_EXPERTS, n), jnp.float32),
    )(x, W, b2)
    return out_t.T
